# baseline (speedup 1.0000x reference)
import functools

import jax
import jax.numpy as jnp
from jax import lax
from jax.experimental import pallas as pl
from jax.experimental.pallas import tpu as pltpu
from jax.experimental.pallas import tpu_sc as plsc

B = 16384
GMF_DIM = 16
MLP_DIM = 64

NC = 2
NS = 16
NW = NC * NS
BPW = B // NW
CHUNK = 128
NCHUNK = BPW // CHUNK


def _sc_body(user_hbm, item_hbm, eug_hbm, eig_hbm, eum_hbm, eim_hbm,
             ug_out, ig_out, um_out, im_out,
             idx_u, idx_i, bu_g, bi_g, bu_m, bi_m, sem):
  wid = lax.axis_index("s") * NC + lax.axis_index("c")
  row0 = wid * NCHUNK
  base = wid * BPW
  pltpu.sync_copy(user_hbm.at[pl.ds(row0, NCHUNK)], idx_u)
  pltpu.sync_copy(item_hbm.at[pl.ds(row0, NCHUNK)], idx_i)
  copies = []
  for j in range(NCHUNK):
    sl = pl.ds(j * CHUNK, CHUNK)
    copies.append(pltpu.async_copy(eug_hbm.at[idx_u.at[j]], bu_g.at[sl], sem))
    copies.append(pltpu.async_copy(eig_hbm.at[idx_i.at[j]], bi_g.at[sl], sem))
    copies.append(pltpu.async_copy(eum_hbm.at[idx_u.at[j]], bu_m.at[sl], sem))
    copies.append(pltpu.async_copy(eim_hbm.at[idx_i.at[j]], bi_m.at[sl], sem))
  for c in copies:
    c.wait()
  out_sl = pl.ds(base, BPW)
  pltpu.sync_copy(bu_g, ug_out.at[out_sl])
  pltpu.sync_copy(bi_g, ig_out.at[out_sl])
  pltpu.sync_copy(bu_m, um_out.at[out_sl])
  pltpu.sync_copy(bi_m, im_out.at[out_sl])


_sc_gather = functools.partial(
    pl.kernel,
    out_type=[
        jax.ShapeDtypeStruct((B, GMF_DIM), jnp.float32),
        jax.ShapeDtypeStruct((B, GMF_DIM), jnp.float32),
        jax.ShapeDtypeStruct((B, MLP_DIM), jnp.float32),
        jax.ShapeDtypeStruct((B, MLP_DIM), jnp.float32),
    ],
    mesh=plsc.VectorSubcoreMesh(
        core_axis_name="c", subcore_axis_name="s",
        num_cores=NC, num_subcores=NS),
    compiler_params=pltpu.CompilerParams(use_tc_tiling_on_sc=False),
    scratch_types=[
        pltpu.VMEM((NCHUNK, CHUNK), jnp.int32),
        pltpu.VMEM((NCHUNK, CHUNK), jnp.int32),
        pltpu.VMEM((BPW, GMF_DIM), jnp.float32),
        pltpu.VMEM((BPW, GMF_DIM), jnp.float32),
        pltpu.VMEM((BPW, MLP_DIM), jnp.float32),
        pltpu.VMEM((BPW, MLP_DIM), jnp.float32),
        pltpu.SemaphoreType.DMA,
    ],
)(_sc_body)


def _tc_body(ug, ig, um, im, w1a, w1b, b1, w2, b2, w3, b3, wpg, wpm, bp, out):
  f32 = jnp.float32
  dot = functools.partial(lax.dot_general, preferred_element_type=f32)
  dn_nt = (((1,), (1,)), ((), ()))
  h = dot(um[...], w1a[...], dn_nt) + dot(im[...], w1b[...], dn_nt) + b1[...]
  h = jnp.maximum(h, 0.0)
  h = jnp.maximum(dot(h, w2[...], dn_nt) + b2[...], 0.0)
  h = jnp.maximum(dot(h, w3[...], dn_nt) + b3[...], 0.0)
  gmf = ug[...] * ig[...]
  pred = dot(gmf, wpg[...], dn_nt) + dot(h, wpm[...], dn_nt) + bp[...]
  out[...] = pred


def _tc_mlp(ug, ig, um, im, w1a, w1b, b1, w2, b2, w3, b3, wpg, wpm, bp):
  ntile = 16
  rows = B // ntile
  full = lambda shape: pl.BlockSpec(shape, lambda i: (0, 0))
  return pl.pallas_call(
      _tc_body,
      grid=(ntile,),
      in_specs=[
          pl.BlockSpec((rows, GMF_DIM), lambda i: (i, 0)),
          pl.BlockSpec((rows, GMF_DIM), lambda i: (i, 0)),
          pl.BlockSpec((rows, MLP_DIM), lambda i: (i, 0)),
          pl.BlockSpec((rows, MLP_DIM), lambda i: (i, 0)),
          full((64, 64)), full((64, 64)), full((1, 64)),
          full((32, 64)), full((1, 32)),
          full((16, 32)), full((1, 16)),
          full((1, 16)), full((1, 16)), full((1, 1)),
      ],
      out_specs=pl.BlockSpec((rows, 1), lambda i: (i, 0)),
      out_shape=jax.ShapeDtypeStruct((B, 1), jnp.float32),
  )(ug, ig, um, im, w1a, w1b, b1, w2, b2, w3, b3, wpg, wpm, bp)


def kernel(user, item, embed_user_GMF, embed_item_GMF, embed_user_MLP,
           embed_item_MLP, W1, b1, W2, b2, W3, b3, Wp, bp):
  user2d = user.astype(jnp.int32).reshape(NW * NCHUNK, CHUNK)
  item2d = item.astype(jnp.int32).reshape(NW * NCHUNK, CHUNK)
  ug, ig, um, im = _sc_gather(
      user2d, item2d, embed_user_GMF, embed_item_GMF,
      embed_user_MLP, embed_item_MLP)
  w1a = W1[:, :MLP_DIM]
  w1b = W1[:, MLP_DIM:]
  wpg = Wp[:, :GMF_DIM]
  wpm = Wp[:, GMF_DIM:]
  out = _tc_mlp(ug, ig, um, im, w1a, w1b, b1.reshape(1, -1),
                W2, b2.reshape(1, -1), W3, b3.reshape(1, -1),
                wpg, wpm, bp.reshape(1, 1))
  return out.reshape(B)


# speedup vs baseline: 1.0269x; 1.0269x over previous
import functools

import jax
import jax.numpy as jnp
from jax import lax
from jax.experimental import pallas as pl
from jax.experimental.pallas import tpu as pltpu
from jax.experimental.pallas import tpu_sc as plsc

B = 16384
GMF_DIM = 16
MLP_DIM = 64
NROWS = 1_000_000

NC = 2
NS = 16
NW = NC * NS
NTILECOL = (NROWS + 127) // 128
COLS_PER_W = (NTILECOL + NW - 1) // NW

W_GMF = 2048
W_MLP = 512
NWIN_GMF = (COLS_PER_W * 128 + W_GMF - 1) // W_GMF
NWIN_MLP = (COLS_PER_W * 128 + W_MLP - 1) // W_MLP
IDX_CHUNK = 2048
NCHUNKS = B // IDX_CHUNK


def _sc_body(user_hbm, item_hbm, tug, tig, tum, tim,
             og, oig, om, oim,
             buf_g, buf_m, ichunk, rlist, blist, rstage, bstage,
             rows, bids, sem):
  wid = lax.axis_index("s") * NC + lax.axis_index("c")
  col0 = wid * COLS_PER_W
  ncols = jnp.minimum(COLS_PER_W, NTILECOL - col0)
  r0 = col0 * 128
  r_end = (col0 + ncols) * 128
  iota = lax.iota(jnp.int32, 16)

  def one_table(idx_hbm, tab_t, out, buf, depth, win, nwin):
    def chunk_body(ch, cnt):
      pltpu.sync_copy(idx_hbm.at[pl.ds(ch * IDX_CHUNK, IDX_CHUNK)], ichunk)

      def scan(v, cnt):
        x = ichunk[pl.ds(v * 16, 16)]
        m = (x >= r0) & (x < r_end)
        n = jnp.max(plsc.all_reduce_population_count(m))

        @pl.when(n > 0)
        def _():
          plsc.store_compressed(rlist.at[pl.ds(cnt, 16)], x, mask=m)
          bvals = ch * IDX_CHUNK + v * 16 + iota
          plsc.store_compressed(blist.at[pl.ds(cnt, 16)], bvals, mask=m)
        return cnt + n
      return lax.fori_loop(0, IDX_CHUNK // 16, scan, cnt)
    cnt = lax.fori_loop(0, NCHUNKS, chunk_body, jnp.int32(0))
    nv = (cnt + 15) // 16

    def win_body(w, _):
      c0 = pl.multiple_of(jnp.minimum(r0 + w * win, r_end - win), 128)
      pltpu.sync_copy(tab_t.at[:, pl.ds(c0, win)], buf)

      def scan_list(v, _):
        lane_ok = (v * 16 + iota) < cnt
        r = rlist[pl.ds(v * 16, 16)]
        b = blist[pl.ds(v * 16, 16)]
        m = (r >= c0) & (r < c0 + win) & lane_ok
        n = jnp.max(plsc.all_reduce_population_count(m))

        @pl.when(n > 0)
        def _():
          plsc.store_compressed(rstage.at[pl.ds(0, 16)], r - c0, mask=m)
          plsc.store_compressed(bstage.at[pl.ds(0, 16)], b, mask=m)
          j = jnp.minimum(iota, n - 1)
          cc = jnp.take(rstage[...], j)
          bids[...] = jnp.take(bstage[...], j)
          for i in range(16):
            ci = jnp.take(cc, jnp.full((16,), i, jnp.int32))
            for g in range(depth // 16):
              vals = plsc.load_gather(buf, [iota + 16 * g, ci])
              rows[i, pl.ds(16 * g, 16)] = vals
          pltpu.async_copy(rows, out.at[bids], sem).wait()
        return 0
      lax.fori_loop(0, nv, scan_list, 0)
      return 0
    lax.fori_loop(0, nwin, win_body, 0)

  one_table(user_hbm, tug, og, buf_g, GMF_DIM, W_GMF, NWIN_GMF)
  one_table(item_hbm, tig, oig, buf_g, GMF_DIM, W_GMF, NWIN_GMF)
  one_table(user_hbm, tum, om, buf_m, MLP_DIM, W_MLP, NWIN_MLP)
  one_table(item_hbm, tim, oim, buf_m, MLP_DIM, W_MLP, NWIN_MLP)


_sc_gather = functools.partial(
    pl.kernel,
    out_type=[
        jax.ShapeDtypeStruct((B, 128), jnp.float32),
        jax.ShapeDtypeStruct((B, 128), jnp.float32),
        jax.ShapeDtypeStruct((B, 128), jnp.float32),
        jax.ShapeDtypeStruct((B, 128), jnp.float32),
    ],
    mesh=plsc.VectorSubcoreMesh(
        core_axis_name="c", subcore_axis_name="s",
        num_cores=NC, num_subcores=NS),
    compiler_params=pltpu.CompilerParams(needs_layout_passes=False),
    scratch_types=[
        pltpu.VMEM((GMF_DIM, W_GMF), jnp.float32),
        pltpu.VMEM((MLP_DIM, W_MLP), jnp.float32),
        pltpu.VMEM((IDX_CHUNK,), jnp.int32),
        pltpu.VMEM((B + 16,), jnp.int32),
        pltpu.VMEM((B + 16,), jnp.int32),
        pltpu.VMEM((16,), jnp.int32),
        pltpu.VMEM((16,), jnp.int32),
        pltpu.VMEM((16, 128), jnp.float32),
        pltpu.VMEM((16,), jnp.int32),
        pltpu.SemaphoreType.DMA,
    ],
)(_sc_body)


def _tc_body(ug, ig, um, im, w1a, w1b, b1, w2, b2, w3, b3, wpg, wpm, bp, out):
  dot = functools.partial(lax.dot_general, preferred_element_type=jnp.float32)
  dn_nt = (((1,), (1,)), ((), ()))
  eu = um[...][:, :MLP_DIM]
  ei = im[...][:, :MLP_DIM]
  h = dot(eu, w1a[...], dn_nt) + dot(ei, w1b[...], dn_nt) + b1[...]
  h = jnp.maximum(h, 0.0)
  h = jnp.maximum(dot(h, w2[...], dn_nt) + b2[...], 0.0)
  h = jnp.maximum(dot(h, w3[...], dn_nt) + b3[...], 0.0)
  gmf = ug[...][:, :GMF_DIM] * ig[...][:, :GMF_DIM]
  pred = dot(gmf, wpg[...], dn_nt) + dot(h, wpm[...], dn_nt) + bp[...]
  out[...] = pred


def _tc_mlp(ug, ig, um, im, w1a, w1b, b1, w2, b2, w3, b3, wpg, wpm, bp):
  ntile = 16
  rows = B // ntile
  full = lambda shape: pl.BlockSpec(shape, lambda i: (0, 0))
  return pl.pallas_call(
      _tc_body,
      grid=(ntile,),
      in_specs=[
          pl.BlockSpec((rows, 128), lambda i: (i, 0)),
          pl.BlockSpec((rows, 128), lambda i: (i, 0)),
          pl.BlockSpec((rows, 128), lambda i: (i, 0)),
          pl.BlockSpec((rows, 128), lambda i: (i, 0)),
          full((64, 64)), full((64, 64)), full((1, 64)),
          full((32, 64)), full((1, 32)),
          full((16, 32)), full((1, 16)),
          full((1, 16)), full((1, 16)), full((1, 1)),
      ],
      out_specs=pl.BlockSpec((rows, 1), lambda i: (i, 0)),
      out_shape=jax.ShapeDtypeStruct((B, 1), jnp.float32),
  )(ug, ig, um, im, w1a, w1b, b1, w2, b2, w3, b3, wpg, wpm, bp)


def kernel(user, item, embed_user_GMF, embed_item_GMF, embed_user_MLP,
           embed_item_MLP, W1, b1, W2, b2, W3, b3, Wp, bp):
  user = user.astype(jnp.int32)
  item = item.astype(jnp.int32)
  ug, ig, um, im = _sc_gather(
      user, item, embed_user_GMF.T, embed_item_GMF.T,
      embed_user_MLP.T, embed_item_MLP.T)
  w1a = W1[:, :MLP_DIM]
  w1b = W1[:, MLP_DIM:]
  wpg = Wp[:, :GMF_DIM]
  wpm = Wp[:, GMF_DIM:]
  out = _tc_mlp(ug, ig, um, im, w1a, w1b, b1.reshape(1, -1),
                W2, b2.reshape(1, -1), W3, b3.reshape(1, -1),
                wpg, wpm, bp.reshape(1, 1))
  return out.reshape(B)


# speedup vs baseline: 3.6506x; 3.5549x over previous
import functools

import jax
import jax.numpy as jnp
from jax import lax
from jax.experimental import pallas as pl
from jax.experimental.pallas import tpu as pltpu
from jax.experimental.pallas import tpu_sc as plsc

B = 16384
GMF_DIM = 16
MLP_DIM = 64
NROWS = 1_000_000

NC = 2
NS = 16
NW = NC * NS
NTILECOL = (NROWS + 127) // 128
COLS_PER_W = (NTILECOL + NW - 1) // NW

W_GMF = 1024
W_MLP = 384
NWIN_GMF = (COLS_PER_W * 128 + W_GMF - 1) // W_GMF
NWIN_MLP = (COLS_PER_W * 128 + W_MLP - 1) // W_MLP
IDX_CHUNK = 2048
NCHUNKS = B // IDX_CHUNK


def _sc_body(user_hbm, item_hbm, tug, tig, tum, tim,
             og, oig, om, oim,
             buf_g, buf_m, ichunk, rlist, blist, rstage, bstage,
             rows_acc, abids, bids, sems, ssem):
  wid = lax.axis_index("s") * NC + lax.axis_index("c")
  col0 = wid * COLS_PER_W
  ncols = jnp.minimum(COLS_PER_W, NTILECOL - col0)
  r0 = col0 * 128
  r_end = (col0 + ncols) * 128
  iota = lax.iota(jnp.int32, 16)

  def build_list(idx_hbm):
    def chunk_body(ch, cnt):
      pltpu.sync_copy(idx_hbm.at[pl.ds(ch * IDX_CHUNK, IDX_CHUNK)], ichunk)

      def scan(v, cnt):
        x = ichunk[pl.ds(v * 16, 16)]
        m = (x >= r0) & (x < r_end)
        n = jnp.max(plsc.all_reduce_population_count(m))

        @pl.when(n > 0)
        def _():
          plsc.store_compressed(rlist.at[pl.ds(cnt, 16)], x, mask=m)
          bvals = ch * IDX_CHUNK + v * 16 + iota
          plsc.store_compressed(blist.at[pl.ds(cnt, 16)], bvals, mask=m)
        return cnt + n
      return lax.fori_loop(0, IDX_CHUNK // 16, scan, cnt)
    return lax.fori_loop(0, NCHUNKS, chunk_body, jnp.int32(0))

  def one_table(tab_t, out, buf2, depth, win, nwin, cnt):
    nv = (cnt + 15) // 16
    ngrp = depth // 16

    def flush(a):
      loaded = abids[pl.ds(0, 16)]
      bids[...] = jnp.where(iota < a, loaded, B)
      pltpu.async_copy(rows_acc, out.at[bids], ssem).wait()

    def win_c0(w):
      return pl.multiple_of(jnp.minimum(r0 + w * win, r_end - win), 128)

    def dma(w):
      p = lax.rem(w, 2)
      return pltpu.make_async_copy(
          tab_t.at[:, pl.ds(win_c0(w), win)], buf2.at[p], sems.at[p])

    dma(0).start()

    def win_body(w, a):
      @pl.when(w + 1 < nwin)
      def _():
        dma(w + 1).start()
      dma(w).wait()
      c0 = win_c0(w)
      buf = buf2.at[lax.rem(w, 2)]

      def scan_list(v, a):
        lane_ok = (v * 16 + iota) < cnt
        r = rlist[pl.ds(v * 16, 16)]
        b = blist[pl.ds(v * 16, 16)]
        m = (r >= c0) & (r < c0 + win) & lane_ok
        n = jnp.max(plsc.all_reduce_population_count(m))

        def append(a):
          plsc.store_compressed(rstage.at[pl.ds(0, 16)], r - c0, mask=m)
          need_flush = a + n > 16

          @pl.when(need_flush)
          def _():
            flush(a)
          a2 = jnp.where(need_flush, 0, a)
          plsc.store_compressed(abids.at[pl.ds(a2, 16)], b, mask=m)
          cc = rstage[...]

          def extract(i, _):
            ci = jnp.take(cc, jnp.zeros((16,), jnp.int32) + i)
            for g in range(ngrp):
              vals = plsc.load_gather(buf, [iota + 16 * g, ci])
              rows_acc[a2 + i, pl.ds(16 * g, 16)] = vals
            return 0
          lax.fori_loop(0, n, extract, 0)
          return a2 + n
        return lax.cond(n > 0, append, lambda a_: a_, a)
      return lax.fori_loop(0, nv, scan_list, a)
    a = lax.fori_loop(0, nwin, win_body, jnp.int32(0))

    @pl.when(a > 0)
    def _():
      flush(a)

  cnt_u = build_list(user_hbm)
  one_table(tug, og, buf_g, GMF_DIM, W_GMF, NWIN_GMF, cnt_u)
  one_table(tum, om, buf_m, MLP_DIM, W_MLP, NWIN_MLP, cnt_u)
  cnt_i = build_list(item_hbm)
  one_table(tig, oig, buf_g, GMF_DIM, W_GMF, NWIN_GMF, cnt_i)
  one_table(tim, oim, buf_m, MLP_DIM, W_MLP, NWIN_MLP, cnt_i)


_sc_gather = functools.partial(
    pl.kernel,
    out_type=[
        jax.ShapeDtypeStruct((B + 16, 128), jnp.float32),
        jax.ShapeDtypeStruct((B + 16, 128), jnp.float32),
        jax.ShapeDtypeStruct((B + 16, 128), jnp.float32),
        jax.ShapeDtypeStruct((B + 16, 128), jnp.float32),
    ],
    mesh=plsc.VectorSubcoreMesh(
        core_axis_name="c", subcore_axis_name="s",
        num_cores=NC, num_subcores=NS),
    compiler_params=pltpu.CompilerParams(needs_layout_passes=False),
    scratch_types=[
        pltpu.VMEM((2, GMF_DIM, W_GMF), jnp.float32),
        pltpu.VMEM((2, MLP_DIM, W_MLP), jnp.float32),
        pltpu.VMEM((IDX_CHUNK,), jnp.int32),
        pltpu.VMEM((B + 16,), jnp.int32),
        pltpu.VMEM((B + 16,), jnp.int32),
        pltpu.VMEM((16,), jnp.int32),
        pltpu.VMEM((16,), jnp.int32),
        pltpu.VMEM((16, 128), jnp.float32),
        pltpu.VMEM((32,), jnp.int32),
        pltpu.VMEM((16,), jnp.int32),
        pltpu.SemaphoreType.DMA((2,)),
        pltpu.SemaphoreType.DMA,
    ],
)(_sc_body)


def _tc_body(ug, ig, um, im, w1a, w1b, b1, w2, b2, w3, b3, wpg, wpm, bp, out):
  dot = functools.partial(lax.dot_general, preferred_element_type=jnp.float32)
  dn_nt = (((1,), (1,)), ((), ()))
  eu = um[...][:, :MLP_DIM]
  ei = im[...][:, :MLP_DIM]
  h = dot(eu, w1a[...], dn_nt) + dot(ei, w1b[...], dn_nt) + b1[...]
  h = jnp.maximum(h, 0.0)
  h = jnp.maximum(dot(h, w2[...], dn_nt) + b2[...], 0.0)
  h = jnp.maximum(dot(h, w3[...], dn_nt) + b3[...], 0.0)
  gmf = ug[...][:, :GMF_DIM] * ig[...][:, :GMF_DIM]
  pred = dot(gmf, wpg[...], dn_nt) + dot(h, wpm[...], dn_nt) + bp[...]
  out[...] = pred


def _tc_mlp(ug, ig, um, im, w1a, w1b, b1, w2, b2, w3, b3, wpg, wpm, bp):
  ntile = 16
  rows = B // ntile
  full = lambda shape: pl.BlockSpec(shape, lambda i: (0, 0))
  inspec = pl.BlockSpec((rows, 128), lambda i: (i, 0))
  return pl.pallas_call(
      _tc_body,
      grid=(ntile,),
      in_specs=[
          inspec, inspec, inspec, inspec,
          full((64, 64)), full((64, 64)), full((1, 64)),
          full((32, 64)), full((1, 32)),
          full((16, 32)), full((1, 16)),
          full((1, 16)), full((1, 16)), full((1, 1)),
      ],
      out_specs=pl.BlockSpec((rows, 1), lambda i: (i, 0)),
      out_shape=jax.ShapeDtypeStruct((B, 1), jnp.float32),
  )(ug, ig, um, im, w1a, w1b, b1, w2, b2, w3, b3, wpg, wpm, bp)


def kernel(user, item, embed_user_GMF, embed_item_GMF, embed_user_MLP,
           embed_item_MLP, W1, b1, W2, b2, W3, b3, Wp, bp):
  user = user.astype(jnp.int32)
  item = item.astype(jnp.int32)
  ug, ig, um, im = _sc_gather(
      user, item, embed_user_GMF.T, embed_item_GMF.T,
      embed_user_MLP.T, embed_item_MLP.T)
  w1a = W1[:, :MLP_DIM]
  w1b = W1[:, MLP_DIM:]
  wpg = Wp[:, :GMF_DIM]
  wpm = Wp[:, GMF_DIM:]
  out = _tc_mlp(ug, ig, um, im, w1a, w1b, b1.reshape(1, -1),
                W2, b2.reshape(1, -1), W3, b3.reshape(1, -1),
                wpg, wpm, bp.reshape(1, 1))
  return out.reshape(B)


# speedup vs baseline: 4.2011x; 1.1508x over previous
import functools

import jax
import jax.numpy as jnp
from jax import lax
from jax.experimental import pallas as pl
from jax.experimental.pallas import tpu as pltpu
from jax.experimental.pallas import tpu_sc as plsc

B = 16384
GMF_DIM = 16
MLP_DIM = 64
NROWS = 1_000_000

NC = 2
NS = 16
NW = NC * NS
NTILECOL = (NROWS + 127) // 128
COLS_PER_W = (NTILECOL + NW - 1) // NW

WIN = 512
NWIN_FULL = 62
NWIN_LAST = 54
TAIL_C0 = 999808
TAIL_C1 = 999936
TAIL_BIN = 54
IDX_CHUNK = 2048
NCHUNKS = B // IDX_CHUNK
NBINS = 64


def _sc_body(user_hbm, item_hbm, tug, tig, tum, tim,
             og, oig, om, oim,
             buf_g, buf_m, ichunk, praw, plist, hist, starts, curs,
             rows_acc, abids, bids, sems, ssem):
  wid = lax.axis_index("s") * NC + lax.axis_index("c")
  islast = wid == NW - 1
  col0 = wid * COLS_PER_W
  ncols = jnp.minimum(COLS_PER_W, NTILECOL - col0)
  r0 = col0 * 128
  r_end = (col0 + ncols) * 128
  nwin = jnp.where(islast, NWIN_LAST, NWIN_FULL)
  iota = lax.iota(jnp.int32, 16)
  zeros16 = jnp.zeros((16,), jnp.int32)
  lane0 = iota == 0

  def scalar_at(ref, i):
    return jnp.max(plsc.load_gather(ref, [zeros16 + i]))

  def build_sorted(idx_hbm):
    def chunk_body(ch, cnt):
      pltpu.sync_copy(idx_hbm.at[pl.ds(ch * IDX_CHUNK, IDX_CHUNK)], ichunk)

      def scan(v, cnt):
        x = ichunk[pl.ds(v * 16, 16)]
        m = (x >= r0) & (x < r_end)
        n = jnp.max(plsc.all_reduce_population_count(m))

        @pl.when(n > 0)
        def _():
          packed = ((x - r0) << 14) | (ch * IDX_CHUNK + v * 16 + iota)
          plsc.store_compressed(praw.at[pl.ds(cnt, 16)], packed, mask=m)
        return cnt + n
      return lax.fori_loop(0, IDX_CHUNK // 16, scan, cnt)
    cnt = lax.fori_loop(0, NCHUNKS, chunk_body, jnp.int32(0))

    for q in range(16 * 128 // 16):
      hist[pl.ds(16 * q, 16)] = zeros16
    ones16 = zeros16 + 1
    lanebase = iota * 128

    def hist_body(t, _):
      x = praw[pl.ds(16 * t, 16)]
      valid = (16 * t + iota) < cnt
      bins = jnp.where(valid, x >> 23, 0) & (NBINS - 1)
      h = plsc.load_gather(hist, [lanebase + bins])
      plsc.store_scatter(hist, [lanebase + bins], h + 1, mask=valid)
      return 0
    lax.fori_loop(0, (cnt + 15) // 16, hist_body, 0)

    starts[pl.ds(0, 16)] = zeros16
    run = jnp.int32(0)
    for q in range(NBINS // 16):
      tot = jnp.zeros((16,), jnp.int32)
      for r in range(16):
        tot = tot + hist[pl.ds(r * 128 + 16 * q, 16)]
      c = plsc.cumsum(tot)
      starts[pl.ds(16 * q + 1, 16)] = c + run
      run = run + jnp.max(c)
    for q in range(NBINS // 16):
      run = starts[pl.ds(16 * q, 16)]
      for r in range(16):
        curs[pl.ds(r * 128 + 16 * q, 16)] = run
        run = run + hist[pl.ds(r * 128 + 16 * q, 16)]

    def place_body(t, _):
      x = praw[pl.ds(16 * t, 16)]
      valid = (16 * t + iota) < cnt
      bins = jnp.where(valid, x >> 23, 0) & (NBINS - 1)
      pos = plsc.load_gather(curs, [lanebase + bins])
      plsc.store_scatter(plist, [pos], x, mask=valid)
      plsc.store_scatter(curs, [lanebase + bins], pos + 1, mask=valid)
      return 0
    lax.fori_loop(0, (cnt + 15) // 16, place_body, 0)

  def flush(a):
    loaded = abids[pl.ds(0, 16)]
    bids[...] = jnp.where(iota < a, loaded, B + wid)
    pltpu.async_copy(rows_acc, out_ref[0].at[bids], ssem).wait()

  def stream_pass(tab_t, out, buf2, depth):
    out_ref[0] = out
    ngrp = depth // 16

    def make_chunks(buf, woff, cs, ce):
      def chunk_body(t, a):
        base = cs + 16 * t
        x16 = plist[pl.ds(base, 16)]
        n = jnp.minimum(16, ce - base)
        valid = iota < n
        col = (x16 >> 14) - woff
        b = x16 & 16383
        need_flush = a + n > 16

        @pl.when(need_flush)
        def _():
          flush(a)
        a2 = jnp.where(need_flush, 0, a)
        plsc.store_compressed(abids.at[pl.ds(a2, 16)], b, mask=valid)

        def extract(i, _):
          ci = jnp.take(col, zeros16 + i)
          for g in range(ngrp):
            vals = plsc.load_gather(buf, [iota + 16 * g, ci])
            rows_acc[a2 + i, pl.ds(16 * g, 16)] = vals
          return 0
        lax.fori_loop(0, n, extract, 0)
        return a2 + n
      return chunk_body

    def dma(w):
      p = lax.rem(w, 2)
      c0 = pl.multiple_of(r0 + w * WIN, 128)
      return pltpu.make_async_copy(
          tab_t.at[:, pl.ds(c0, WIN)], buf2.at[p], sems.at[p])

    dma(0).start()

    def win_body(w, a):
      @pl.when(w + 1 < nwin)
      def _():
        dma(w + 1).start()
      dma(w).wait()
      cs = scalar_at(starts, w)
      ce = scalar_at(starts, w + 1)
      body = make_chunks(buf2.at[lax.rem(w, 2)], w * WIN, cs, ce)
      return lax.fori_loop(0, (ce - cs + 15) // 16, body, a)
    a = lax.fori_loop(0, nwin, win_body, jnp.int32(0))

    c0t = pl.multiple_of(jnp.where(islast, TAIL_C0, r0), 128)
    pltpu.sync_copy(tab_t.at[:, pl.ds(c0t, 256)],
                    buf2.at[0, :, pl.ds(0, 256)])
    cs = scalar_at(starts, TAIL_BIN)
    ce = jnp.where(islast, scalar_at(starts, TAIL_BIN + 1), cs)
    body = make_chunks(buf2.at[0], TAIL_BIN * WIN, cs, ce)
    a = lax.fori_loop(0, (ce - cs + 15) // 16, body, a)

    @pl.when(a > 0)
    def _():
      flush(a)

  out_ref = [None]
  build_sorted(user_hbm)
  stream_pass(tug, og, buf_g, GMF_DIM)
  stream_pass(tum, om, buf_m, MLP_DIM)
  build_sorted(item_hbm)
  stream_pass(tig, oig, buf_g, GMF_DIM)
  stream_pass(tim, oim, buf_m, MLP_DIM)


_sc_gather = functools.partial(
    pl.kernel,
    out_type=[
        jax.ShapeDtypeStruct((B + NW, 128), jnp.float32),
        jax.ShapeDtypeStruct((B + NW, 128), jnp.float32),
        jax.ShapeDtypeStruct((B + NW, 128), jnp.float32),
        jax.ShapeDtypeStruct((B + NW, 128), jnp.float32),
    ],
    mesh=plsc.VectorSubcoreMesh(
        core_axis_name="c", subcore_axis_name="s",
        num_cores=NC, num_subcores=NS),
    compiler_params=pltpu.CompilerParams(needs_layout_passes=False),
    scratch_types=[
        pltpu.VMEM((2, GMF_DIM, WIN), jnp.float32),
        pltpu.VMEM((2, MLP_DIM, WIN), jnp.float32),
        pltpu.VMEM((IDX_CHUNK,), jnp.int32),
        pltpu.VMEM((B + 32,), jnp.int32),
        pltpu.VMEM((B + 32,), jnp.int32),
        pltpu.VMEM((2048,), jnp.int32),
        pltpu.VMEM((NBINS + 17,), jnp.int32),
        pltpu.VMEM((2048,), jnp.int32),
        pltpu.VMEM((16, 128), jnp.float32),
        pltpu.VMEM((32,), jnp.int32),
        pltpu.VMEM((16,), jnp.int32),
        pltpu.SemaphoreType.DMA((2,)),
        pltpu.SemaphoreType.DMA,
    ],
)(_sc_body)


def _tc_body(ug, ig, um, im, w1a, w1b, b1, w2, b2, w3, b3, wpg, wpm, bp, out):
  dot = functools.partial(lax.dot_general, preferred_element_type=jnp.float32)
  dn_nt = (((1,), (1,)), ((), ()))
  eu = um[...][:, :MLP_DIM]
  ei = im[...][:, :MLP_DIM]
  h = dot(eu, w1a[...], dn_nt) + dot(ei, w1b[...], dn_nt) + b1[...]
  h = jnp.maximum(h, 0.0)
  h = jnp.maximum(dot(h, w2[...], dn_nt) + b2[...], 0.0)
  h = jnp.maximum(dot(h, w3[...], dn_nt) + b3[...], 0.0)
  gmf = ug[...][:, :GMF_DIM] * ig[...][:, :GMF_DIM]
  pred = dot(gmf, wpg[...], dn_nt) + dot(h, wpm[...], dn_nt) + bp[...]
  out[...] = pred


def _tc_mlp(ug, ig, um, im, w1a, w1b, b1, w2, b2, w3, b3, wpg, wpm, bp):
  ntile = 16
  rows = B // ntile
  full = lambda shape: pl.BlockSpec(shape, lambda i: (0, 0))
  inspec = pl.BlockSpec((rows, 128), lambda i: (i, 0))
  return pl.pallas_call(
      _tc_body,
      grid=(ntile,),
      in_specs=[
          inspec, inspec, inspec, inspec,
          full((64, 64)), full((64, 64)), full((1, 64)),
          full((32, 64)), full((1, 32)),
          full((16, 32)), full((1, 16)),
          full((1, 16)), full((1, 16)), full((1, 1)),
      ],
      out_specs=pl.BlockSpec((rows, 1), lambda i: (i, 0)),
      out_shape=jax.ShapeDtypeStruct((B, 1), jnp.float32),
  )(ug, ig, um, im, w1a, w1b, b1, w2, b2, w3, b3, wpg, wpm, bp)


def kernel(user, item, embed_user_GMF, embed_item_GMF, embed_user_MLP,
           embed_item_MLP, W1, b1, W2, b2, W3, b3, Wp, bp):
  user = user.astype(jnp.int32)
  item = item.astype(jnp.int32)
  ug, ig, um, im = _sc_gather(
      user, item, embed_user_GMF.T, embed_item_GMF.T,
      embed_user_MLP.T, embed_item_MLP.T)
  w1a = W1[:, :MLP_DIM]
  w1b = W1[:, MLP_DIM:]
  wpg = Wp[:, :GMF_DIM]
  wpm = Wp[:, GMF_DIM:]
  out = _tc_mlp(ug, ig, um, im, w1a, w1b, b1.reshape(1, -1),
                W2, b2.reshape(1, -1), W3, b3.reshape(1, -1),
                wpg, wpm, bp.reshape(1, 1))
  return out.reshape(B)
